# baseline (device time: 89650 ns/iter reference)
import jax
import jax.numpy as jnp
from jax import lax
from jax.experimental import pallas as pl
from jax.experimental.pallas import tpu as pltpu

N_ROWS = 4096
N_COLS = 2048
PART = N_ROWS // 4
K = 8
R = PART // K
DY = 2
DX = 5
KY = K + DY


def kernel(x, pi):
    def body(x_ref, pi_ref, out_ref, xin, xbf, load_sems, ysend_sems,
             yrecv_sems, xs1, xr1, zs1, zr1, xs2, xr2, zs2, zr2):
        my_x = lax.axis_index("x")
        my_y = lax.axis_index("y")
        my_z = lax.axis_index("z")
        dst_y = pi_ref[my_y]
        b = my_z % 2
        zp = my_z + 1 - 2 * b
        p_me = (2 * my_x + b) * PART
        p_x = (2 * (1 - my_x) + b) * PART
        p_z = (2 * my_x + (1 - b)) * PART
        p_diag = (2 * (1 - my_x) + (1 - b)) * PART

        barrier = pltpu.get_barrier_semaphore()
        for dev in ((my_x, 1 - my_y, my_z), (1 - my_x, my_y, my_z),
                    (my_x, my_y, zp)):
            pl.semaphore_signal(
                barrier, inc=1, device_id=dev,
                device_id_type=pl.DeviceIdType.MESH,
            )
        pl.semaphore_wait(barrier, 3)

        def load(k, start):
            return pltpu.make_async_copy(
                x_ref.at[0, pl.ds(start, R), :], xin.at[k % 2],
                load_sems.at[k % 2],
            )

        @pl.when(dst_y == my_y)
        def _identity():
            for k in range(N_ROWS // R):
                load(k, k * R).start()
                load(k, k * R).wait()
                xbf[k % 2] = xin[k % 2].astype(jnp.bfloat16)
                st = pltpu.make_async_copy(
                    xbf.at[k % 2], out_ref.at[0, pl.ds(k * R, R), :],
                    ysend_sems.at[k % 2],
                )
                st.start()
                st.wait()

        @pl.when(dst_y != my_y)
        def _swap():
            ypeer = (my_x, dst_y, my_z)
            xpeer = (1 - my_x, my_y, my_z)
            zpart = (my_x, my_y, zp)

            def out_at(base, k):
                return out_ref.at[0, pl.ds(base + k * R, R), :]

            def rdma(base, k, send_sem, recv_sem, dev):
                return pltpu.make_async_remote_copy(
                    src_ref=out_at(base, k), dst_ref=out_at(base, k),
                    send_sem=send_sem, recv_sem=recv_sem,
                    device_id=dev, device_id_type=pl.DeviceIdType.MESH,
                )

            def ybase(j):
                return p_me + j * R if j < K else p_diag + (j - K) * R

            def y_rdma(j):
                return pltpu.make_async_remote_copy(
                    src_ref=xbf.at[j % 2],
                    dst_ref=out_ref.at[0, pl.ds(ybase(j), R), :],
                    send_sem=ysend_sems.at[j % 2],
                    recv_sem=yrecv_sems.at[j],
                    device_id=ypeer,
                    device_id_type=pl.DeviceIdType.MESH,
                )

            def fx1(k):
                return rdma(p_me, k, xs1.at[k], xr1.at[k], xpeer)

            def fz1(k):
                return rdma(p_me, k, zs1.at[k], zr1.at[k], zpart)

            def rx1(k):
                return rdma(p_x, k, xs1.at[k], xr1.at[k], xpeer)

            def rz1(k):
                return rdma(p_z, k, zs1.at[k], zr1.at[k], zpart)

            def fx2(k):
                return rdma(p_z, k, xs2.at[k - DY], xr2.at[k - DY], xpeer)

            def fz2(k):
                return rdma(p_x, k, zs2.at[k - DX], zr2.at[k - DX], zpart)

            def rx2(k):
                return rdma(p_diag, k, xs2.at[k - DY], xr2.at[k - DY], xpeer)

            def rz2(k):
                return rdma(p_diag, k, zs2.at[k - DX], zr2.at[k - DX], zpart)

            load(0, ybase(0)).start()
            load(1, ybase(1)).start()

            for j in range(KY):
                slot = j % 2
                load(j, ybase(j)).wait()
                if j >= 2:
                    y_rdma(j - 2).wait_send()
                xbf[slot] = xin[slot].astype(jnp.bfloat16)
                y_rdma(j).start()
                if j + 2 < KY:
                    load(j + 2, ybase(j + 2)).start()
                if 1 <= j <= K:
                    y_rdma(j - 1).wait_recv()
                    fx1(j - 1).start()
                    fz1(j - 1).start()
                k2 = j - 3
                if DY <= k2 < DX:
                    rz1(k2).wait_recv()
                    fx2(k2).start()
                elif DX <= k2 < K:
                    rx1(k2).wait_recv()
                    fz2(k2).start()
            for k2 in range(KY - 3, K):
                rx1(k2).wait_recv()
                fz2(k2).start()
            y_rdma(KY - 2).wait_send()
            y_rdma(KY - 1).wait_send()

            for j in range(K, KY):
                y_rdma(j).wait_recv()
            for k in range(DX):
                rx1(k).wait_recv()
            for k in range(DY):
                rz1(k).wait_recv()
            for k in range(DX, K):
                rz1(k).wait_recv()
            for k in range(DY, DX):
                rx2(k).wait_recv()
            for k in range(DX, K):
                rz2(k).wait_recv()
            for k in range(K):
                fx1(k).wait_send()
                fz1(k).wait_send()
            for k in range(DY, DX):
                fx2(k).wait_send()
            for k in range(DX, K):
                fz2(k).wait_send()

    return pl.pallas_call(
        body,
        out_shape=jax.ShapeDtypeStruct(x.shape, jnp.bfloat16),
        in_specs=[
            pl.BlockSpec(memory_space=pltpu.MemorySpace.HBM),
            pl.BlockSpec(memory_space=pltpu.SMEM),
        ],
        out_specs=pl.BlockSpec(memory_space=pltpu.MemorySpace.HBM),
        scratch_shapes=[
            pltpu.VMEM((2, R, N_COLS), jnp.float32),
            pltpu.VMEM((2, R, N_COLS), jnp.bfloat16),
            pltpu.SemaphoreType.DMA((2,)),
            pltpu.SemaphoreType.DMA((2,)),
            pltpu.SemaphoreType.DMA((KY,)),
            pltpu.SemaphoreType.DMA((K,)),
            pltpu.SemaphoreType.DMA((K,)),
            pltpu.SemaphoreType.DMA((K,)),
            pltpu.SemaphoreType.DMA((K,)),
            pltpu.SemaphoreType.DMA((DX - DY,)),
            pltpu.SemaphoreType.DMA((DX - DY,)),
            pltpu.SemaphoreType.DMA((K - DX,)),
            pltpu.SemaphoreType.DMA((K - DX,)),
        ],
        compiler_params=pltpu.CompilerParams(collective_id=0),
    )(x, pi)


# device time: 81252 ns/iter; 1.1034x vs baseline; 1.1034x over previous
import jax
import jax.numpy as jnp
from jax import lax
from jax.experimental import pallas as pl
from jax.experimental.pallas import tpu as pltpu

N_ROWS = 4096
N_COLS = 2048
PART = N_ROWS // 4
K = 8
R = PART // K
DY = 4
DX = 6
KY = K + DY


def kernel(x, pi):
    def body(x_ref, pi_ref, out_ref, xin, xbf, load_sems, ysend_sems,
             yrecv_sems, xs1, xr1, zs1, zr1, xs2, xr2, zs2, zr2):
        my_x = lax.axis_index("x")
        my_y = lax.axis_index("y")
        my_z = lax.axis_index("z")
        dst_y = pi_ref[my_y]
        b = my_z % 2
        zp = my_z + 1 - 2 * b
        p_me = (2 * my_x + b) * PART
        p_x = (2 * (1 - my_x) + b) * PART
        p_z = (2 * my_x + (1 - b)) * PART
        p_diag = (2 * (1 - my_x) + (1 - b)) * PART

        barrier = pltpu.get_barrier_semaphore()
        for dev in ((my_x, 1 - my_y, my_z), (1 - my_x, my_y, my_z),
                    (my_x, my_y, zp)):
            pl.semaphore_signal(
                barrier, inc=1, device_id=dev,
                device_id_type=pl.DeviceIdType.MESH,
            )
        pl.semaphore_wait(barrier, 3)

        def load(k, start):
            return pltpu.make_async_copy(
                x_ref.at[0, pl.ds(start, R), :], xin.at[k % 2],
                load_sems.at[k % 2],
            )

        @pl.when(dst_y == my_y)
        def _identity():
            for k in range(N_ROWS // R):
                load(k, k * R).start()
                load(k, k * R).wait()
                xbf[k % 2] = xin[k % 2].astype(jnp.bfloat16)
                st = pltpu.make_async_copy(
                    xbf.at[k % 2], out_ref.at[0, pl.ds(k * R, R), :],
                    ysend_sems.at[k % 2],
                )
                st.start()
                st.wait()

        @pl.when(dst_y != my_y)
        def _swap():
            ypeer = (my_x, dst_y, my_z)
            xpeer = (1 - my_x, my_y, my_z)
            zpart = (my_x, my_y, zp)

            def out_at(base, k):
                return out_ref.at[0, pl.ds(base + k * R, R), :]

            def rdma(base, k, send_sem, recv_sem, dev):
                return pltpu.make_async_remote_copy(
                    src_ref=out_at(base, k), dst_ref=out_at(base, k),
                    send_sem=send_sem, recv_sem=recv_sem,
                    device_id=dev, device_id_type=pl.DeviceIdType.MESH,
                )

            def ybase(j):
                return p_me + j * R if j < K else p_diag + (j - K) * R

            def y_rdma(j):
                return pltpu.make_async_remote_copy(
                    src_ref=xbf.at[j % 2],
                    dst_ref=out_ref.at[0, pl.ds(ybase(j), R), :],
                    send_sem=ysend_sems.at[j % 2],
                    recv_sem=yrecv_sems.at[j],
                    device_id=ypeer,
                    device_id_type=pl.DeviceIdType.MESH,
                )

            def fx1(k):
                return rdma(p_me, k, xs1.at[k], xr1.at[k], xpeer)

            def fz1(k):
                return rdma(p_me, k, zs1.at[k], zr1.at[k], zpart)

            def rx1(k):
                return rdma(p_x, k, xs1.at[k], xr1.at[k], xpeer)

            def rz1(k):
                return rdma(p_z, k, zs1.at[k], zr1.at[k], zpart)

            def fx2(k):
                return rdma(p_z, k, xs2.at[k - DY], xr2.at[k - DY], xpeer)

            def fz2(k):
                return rdma(p_x, k, zs2.at[k - DX], zr2.at[k - DX], zpart)

            def rx2(k):
                return rdma(p_diag, k, xs2.at[k - DY], xr2.at[k - DY], xpeer)

            def rz2(k):
                return rdma(p_diag, k, zs2.at[k - DX], zr2.at[k - DX], zpart)

            load(0, ybase(0)).start()
            load(1, ybase(1)).start()

            for j in range(KY):
                slot = j % 2
                load(j, ybase(j)).wait()
                if j >= 2:
                    y_rdma(j - 2).wait_send()
                xbf[slot] = xin[slot].astype(jnp.bfloat16)
                y_rdma(j).start()
                if j + 2 < KY:
                    load(j + 2, ybase(j + 2)).start()
                if 1 <= j <= K:
                    y_rdma(j - 1).wait_recv()
                    fx1(j - 1).start()
                    fz1(j - 1).start()

            for k in range(DY, DX):
                rz1(k).wait_recv()
                fx2(k).start()
            for k in range(DX, K):
                rx1(k).wait_recv()
                fz2(k).start()
            y_rdma(KY - 2).wait_send()
            y_rdma(KY - 1).wait_send()

            for j in range(K, KY):
                y_rdma(j).wait_recv()
            for k in range(DX):
                rx1(k).wait_recv()
            for k in range(DY):
                rz1(k).wait_recv()
            for k in range(DX, K):
                rz1(k).wait_recv()
            for k in range(DY, DX):
                rx2(k).wait_recv()
            for k in range(DX, K):
                rz2(k).wait_recv()
            for k in range(K):
                fx1(k).wait_send()
                fz1(k).wait_send()
            for k in range(DY, DX):
                fx2(k).wait_send()
            for k in range(DX, K):
                fz2(k).wait_send()

    return pl.pallas_call(
        body,
        out_shape=jax.ShapeDtypeStruct(x.shape, jnp.bfloat16),
        in_specs=[
            pl.BlockSpec(memory_space=pltpu.MemorySpace.HBM),
            pl.BlockSpec(memory_space=pltpu.SMEM),
        ],
        out_specs=pl.BlockSpec(memory_space=pltpu.MemorySpace.HBM),
        scratch_shapes=[
            pltpu.VMEM((2, R, N_COLS), jnp.float32),
            pltpu.VMEM((2, R, N_COLS), jnp.bfloat16),
            pltpu.SemaphoreType.DMA((2,)),
            pltpu.SemaphoreType.DMA((2,)),
            pltpu.SemaphoreType.DMA((KY,)),
            pltpu.SemaphoreType.DMA((K,)),
            pltpu.SemaphoreType.DMA((K,)),
            pltpu.SemaphoreType.DMA((K,)),
            pltpu.SemaphoreType.DMA((K,)),
            pltpu.SemaphoreType.DMA((DX - DY,)),
            pltpu.SemaphoreType.DMA((DX - DY,)),
            pltpu.SemaphoreType.DMA((K - DX,)),
            pltpu.SemaphoreType.DMA((K - DX,)),
        ],
        compiler_params=pltpu.CompilerParams(collective_id=0),
    )(x, pi)
